# Initial kernel scaffold; baseline (speedup 1.0000x reference)
import jax
import jax.numpy as jnp
from jax.experimental import pallas as pl


def kernel(x, table_year, table_month, table_day, table_hour, table_weekday):
    raise NotImplementedError("write your pallas kernel here")


# speedup vs baseline: 2.0384x; 2.0384x over previous
import functools

import jax
import jax.numpy as jnp
import numpy as np
from jax import lax
from jax.experimental import pallas as pl
from jax.experimental.pallas import tpu as pltpu
from jax.experimental.pallas import tpu_sc as plsc

B = 16384
D = 32
NUM_F = 5
_SIZES = (11, 12, 31, 24, 6)
_CUM = tuple(int(v) for v in np.concatenate([[0], np.cumsum(_SIZES)[:-1]]))
V_ALL = sum(_SIZES)

NC, NS, L = 2, 16, 16
NW = NC * NS
PW = B * NUM_F // NW
CHUNK = 128
NCHUNK = PW // CHUNK

_OFF_PHASES = np.array(
    [[_CUM[(phase + l) % NUM_F] for l in range(L)] for phase in range(NUM_F)],
    dtype=np.int32,
)


def _body(x_hbm, off_hbm, tall_hbm, out_hbm, xv, offv, idxv, outv, sem):
    wid = lax.axis_index("s") * NC + lax.axis_index("c")
    pltpu.sync_copy(x_hbm.at[wid], xv)
    pltpu.sync_copy(off_hbm, offv)
    vregs_per_row = CHUNK // L
    for i in range(NCHUNK):
        for j in range(vregs_per_row):
            v = i * vregs_per_row + j
            off = offv[v % NUM_F, :]
            sl = pl.ds(j * L, L)
            idxv[i, sl] = xv[i, sl] + off
    copies = [
        pltpu.make_async_copy(
            tall_hbm.at[idxv.at[i]],
            outv.at[pl.ds(i * CHUNK, CHUNK)],
            sem,
        )
        for i in range(NCHUNK)
    ]
    for c in copies:
        c.start()
    for c in copies:
        c.wait()
    pltpu.sync_copy(outv, out_hbm.at[pl.ds(wid * PW, PW)])


@jax.jit
def _embed(x3, off, tall):
    mesh = plsc.VectorSubcoreMesh(core_axis_name="c", subcore_axis_name="s")
    run = functools.partial(
        pl.kernel,
        mesh=mesh,
        out_type=jax.ShapeDtypeStruct((B * NUM_F, D), jnp.float32),
        scratch_types=[
            pltpu.VMEM((NCHUNK, CHUNK), jnp.int32),
            pltpu.VMEM((NUM_F, L), jnp.int32),
            pltpu.VMEM((NCHUNK, CHUNK), jnp.int32),
            pltpu.VMEM((PW, D), jnp.float32),
            pltpu.SemaphoreType.DMA,
        ],
        compiler_params=pltpu.CompilerParams(use_tc_tiling_on_sc=False),
    )(_body)
    return run(x3, off, tall)


def kernel(x, table_year, table_month, table_day, table_hour, table_weekday):
    tall = jnp.concatenate(
        [table_year, table_month, table_day, table_hour, table_weekday], axis=0
    )
    x3 = x.astype(jnp.int32).reshape(NW, NCHUNK, CHUNK)
    out = _embed(x3, jnp.asarray(_OFF_PHASES), tall)
    return out.reshape(B, NUM_F * D)


# speedup vs baseline: 4.8362x; 2.3725x over previous
import functools

import jax
import jax.numpy as jnp
import numpy as np
from jax import lax
from jax.experimental import pallas as pl
from jax.experimental.pallas import tpu as pltpu
from jax.experimental.pallas import tpu_sc as plsc

B = 16384
D = 32
NUM_F = 5
_SIZES = (11, 12, 31, 24, 6)
_CUM = tuple(int(v) for v in np.concatenate([[0], np.cumsum(_SIZES)[:-1]]))
V_ALL = sum(_SIZES)

NC, NS, L = 2, 16, 16
NW = NC * NS
PW = B * NUM_F // NW
CHUNK = 128
NCHUNK = PW // CHUNK

_OFF_PHASES = np.array(
    [[_CUM[(phase + l) % NUM_F] for l in range(L)] for phase in range(NUM_F)],
    dtype=np.int32,
)


def _body(x_hbm, off_hbm, tall_hbm, out_hbm, xv, offv, tallv, idxv, outv, sem):
    wid = lax.axis_index("s") * NC + lax.axis_index("c")
    pltpu.sync_copy(x_hbm.at[wid], xv)
    pltpu.sync_copy(off_hbm, offv)
    sid = lax.axis_index("s")
    @pl.when(sid == 0)
    def _():
        pltpu.sync_copy(tall_hbm, tallv)
    plsc.subcore_barrier()
    vregs_per_row = CHUNK // L
    for i in range(NCHUNK):
        for j in range(vregs_per_row):
            v = i * vregs_per_row + j
            off = offv[v % NUM_F, :]
            sl = pl.ds(j * L, L)
            idxv[i, sl] = xv[i, sl] + off
    copies = [
        pltpu.make_async_copy(
            tallv.at[idxv.at[i]],
            outv.at[pl.ds(i * CHUNK, CHUNK)],
            sem,
        )
        for i in range(NCHUNK)
    ]
    for c in copies:
        c.start()
    for c in copies:
        c.wait()
    pltpu.sync_copy(outv, out_hbm.at[pl.ds(wid * PW, PW)])


@jax.jit
def _embed(x3, off, tall):
    mesh = plsc.VectorSubcoreMesh(core_axis_name="c", subcore_axis_name="s")
    run = functools.partial(
        pl.kernel,
        mesh=mesh,
        out_type=jax.ShapeDtypeStruct((B * NUM_F, D), jnp.float32),
        scratch_types=[
            pltpu.VMEM((NCHUNK, CHUNK), jnp.int32),
            pltpu.VMEM((NUM_F, L), jnp.int32),
            pltpu.VMEM_SHARED((V_ALL, D), jnp.float32),
            pltpu.VMEM((NCHUNK, CHUNK), jnp.int32),
            pltpu.VMEM((PW, D), jnp.float32),
            pltpu.SemaphoreType.DMA,
        ],
        compiler_params=pltpu.CompilerParams(use_tc_tiling_on_sc=False),
    )(_body)
    return run(x3, off, tall)


def kernel(x, table_year, table_month, table_day, table_hour, table_weekday):
    tall = jnp.concatenate(
        [table_year, table_month, table_day, table_hour, table_weekday], axis=0
    )
    x3 = x.astype(jnp.int32).reshape(NW, NCHUNK, CHUNK)
    out = _embed(x3, jnp.asarray(_OFF_PHASES), tall)
    return out.reshape(B, NUM_F * D)
